# baseline (speedup 1.0000x reference)
import functools

import jax
import jax.numpy as jnp
from jax.experimental import pallas as pl
from jax.experimental.pallas import tpu as pltpu

_VMEM_LIMIT = 56 * 1024 * 1024


def _cdiv(a, b):
    return (a + b - 1) // b


def _pad_ring(y):
    return jnp.pad(y, ((0, 0), (1, 1), (1, 1), (0, 0)))


def _conv3x3(x, b_ref, s_ref, Ho, Wo):
    nb, _, _, C = x.shape
    taps = [x[:, di:di + Ho, dj:dj + Wo, :]
            for di in range(3) for dj in range(3)]
    a = jnp.concatenate(taps, axis=3).reshape(nb * Ho * Wo, 9 * C)
    y = jnp.dot(a, b_ref[...], preferred_element_type=jnp.float32)
    return (y + s_ref[...]).reshape(nb, Ho, Wo, y.shape[-1])


def _plain_block_kernel(x_ref, b1_ref, s1_ref, b2_ref, s2_ref, o_ref,
                        *, Ho, Wo):
    x = x_ref[...]
    y1 = jnp.maximum(_conv3x3(x, b1_ref, s1_ref, Ho, Wo), 0.0)
    mid = _pad_ring(y1.astype(jnp.bfloat16))
    y2 = _conv3x3(mid, b2_ref, s2_ref, Ho, Wo)
    y2 = y2 + x[:, 1:-1, 1:-1, :].astype(jnp.float32)
    y2 = jnp.maximum(y2, 0.0)
    o_ref[...] = _pad_ring(y2.astype(o_ref.dtype))


def _down_block_kernel(x_ref, b1_ref, s1_ref, bd_ref, sd_ref, b2_ref,
                       s2_ref, o_ref, *, C, Ho, Wo):
    x = x_ref[...]
    nb = x.shape[0]
    taps = []
    for di in range(3):
        p, a = di % 2, di // 2
        for dj in range(3):
            q, bb = dj % 2, dj // 2
            taps.append(x[:, a:a + Ho, p, bb:bb + Wo, q * C:(q + 1) * C])
    a_mat = jnp.concatenate(taps, axis=3).reshape(nb * Ho * Wo, 9 * C)
    y1 = jnp.dot(a_mat, b1_ref[...], preferred_element_type=jnp.float32)
    y1 = jnp.maximum(y1 + s1_ref[...], 0.0)
    mid = _pad_ring(y1.reshape(nb, Ho, Wo, y1.shape[-1])
                    .astype(jnp.bfloat16))

    xph = x[:, :Ho, 1, :Wo, C:2 * C]
    yd = jnp.dot(xph.reshape(nb * Ho * Wo, C), bd_ref[...],
                 preferred_element_type=jnp.float32)
    yd = (yd + sd_ref[...]).astype(jnp.bfloat16)

    y2 = _conv3x3(mid, b2_ref, s2_ref, Ho, Wo)
    y2 = y2 + yd.reshape(y2.shape).astype(jnp.float32)
    y2 = jnp.maximum(y2, 0.0)
    o_ref[...] = _pad_ring(y2.astype(o_ref.dtype))


def _stem_pool_kernel(q_ref, b_ref, s_ref, o_ref, *, Ho, Wo):
    q = q_ref[...]
    nb = q.shape[0]
    shift = s_ref[...]
    taps = [q[:, a:a + Ho, b:b + Wo, :]
            for a in range(3) for b in range(3)]
    a_mat = jnp.concatenate(taps, axis=3).reshape(nb * Ho * Wo, 9 * 64)
    y4 = jnp.dot(a_mat, b_ref[...], preferred_element_type=jnp.float32)
    ys = [y4[:, k * 64:(k + 1) * 64].reshape(nb, Ho, Wo, 64)
          for k in range(4)]
    y00, y01, y10, y11 = ys

    def sh_i(y):
        z = jnp.full_like(y[:, :1], -jnp.inf)
        return jnp.concatenate([z, y[:, :-1]], axis=1)

    def sh_j(y):
        z = jnp.full_like(y[:, :, :1], -jnp.inf)
        return jnp.concatenate([z, y[:, :, :-1]], axis=2)

    c0 = jnp.maximum(jnp.maximum(y00, y01), sh_j(y01))
    c1 = jnp.maximum(jnp.maximum(y10, y11), sh_j(y11))
    m = jnp.maximum(jnp.maximum(c0, c1), sh_i(c1))
    m = jnp.maximum(m + shift, 0.0)
    o_ref[...] = _pad_ring(m.astype(o_ref.dtype))


def _head_kernel(x_ref, b_ref, s_ref, o_ref):
    x = x_ref[...][:, 1:-1, 1:-1, :].astype(jnp.float32)
    xm = jnp.mean(x, axis=(1, 2))
    y = jnp.dot(xm.astype(jnp.bfloat16), b_ref[...],
                preferred_element_type=jnp.float32)
    o_ref[...] = y + s_ref[...]


def _compiler_params(n_par):
    return pltpu.CompilerParams(
        dimension_semantics=("parallel",) * n_par,
        vmem_limit_bytes=_VMEM_LIMIT)


def plain_block(xp, b1, s1, b2, s2, *, nb):
    N, Hp, Wp, C = xp.shape
    Ho, Wo = Hp - 2, Wp - 2
    Cout = b2.shape[1]
    return pl.pallas_call(
        functools.partial(_plain_block_kernel, Ho=Ho, Wo=Wo),
        out_shape=jax.ShapeDtypeStruct((N, Ho + 2, Wo + 2, Cout),
                                       jnp.bfloat16),
        grid=(N // nb,),
        in_specs=[
            pl.BlockSpec((nb, Hp, Wp, C), lambda i: (i, 0, 0, 0)),
            pl.BlockSpec((9 * C, Cout), lambda i: (0, 0)),
            pl.BlockSpec((1, Cout), lambda i: (0, 0)),
            pl.BlockSpec((9 * Cout, Cout), lambda i: (0, 0)),
            pl.BlockSpec((1, Cout), lambda i: (0, 0)),
        ],
        out_specs=pl.BlockSpec((nb, Ho + 2, Wo + 2, Cout),
                               lambda i: (i, 0, 0, 0)),
        compiler_params=_compiler_params(1),
    )(xp, b1, s1, b2, s2)


def down_block(xp, b1, s1, bd, sd, b2, s2, *, nb):
    N, Hp, Wp, C = xp.shape
    Ho, Wo = (Hp - 2) // 2, (Wp - 2) // 2
    Cout = b2.shape[1]
    xf = xp.reshape(N, Hp // 2, 2, Wp // 2, 2 * C)
    return pl.pallas_call(
        functools.partial(_down_block_kernel, C=C, Ho=Ho, Wo=Wo),
        out_shape=jax.ShapeDtypeStruct((N, Ho + 2, Wo + 2, Cout),
                                       jnp.bfloat16),
        grid=(N // nb,),
        in_specs=[
            pl.BlockSpec((nb, Hp // 2, 2, Wp // 2, 2 * C),
                         lambda i: (i, 0, 0, 0, 0)),
            pl.BlockSpec((9 * C, Cout), lambda i: (0, 0)),
            pl.BlockSpec((1, Cout), lambda i: (0, 0)),
            pl.BlockSpec((C, Cout), lambda i: (0, 0)),
            pl.BlockSpec((1, Cout), lambda i: (0, 0)),
            pl.BlockSpec((9 * Cout, Cout), lambda i: (0, 0)),
            pl.BlockSpec((1, Cout), lambda i: (0, 0)),
        ],
        out_specs=pl.BlockSpec((nb, Ho + 2, Wo + 2, Cout),
                               lambda i: (i, 0, 0, 0)),
        compiler_params=_compiler_params(1),
    )(xf, b1, s1, bd, sd, b2, s2)


def _stem_phase_weights(b):
    bp = jnp.concatenate([b, jnp.zeros((1, b.shape[1]), b.dtype)], axis=0)
    mats = []
    for r in (0, 1):
        for s in (0, 1):
            rows = []
            for a in range(3):
                for bb in range(3):
                    for gh in range(4):
                        for gw in range(4):
                            for c in range(4):
                                di = 4 * a + gh - 2 * r
                                dj = 4 * bb + gw - 2 * s
                                if 0 <= di < 7 and 0 <= dj < 7 and c < 3:
                                    rows.append((di * 7 + dj) * 3 + c)
                                else:
                                    rows.append(147)
            mats.append(bp[jnp.array(rows)])
    return mats


def stem_conv_pool(image, b, shift, *, nb):
    N = image.shape[0]
    Ho, Wo = 56, 56
    xp = jnp.pad(image.astype(jnp.bfloat16), ((0, 0), (0, 1), (3, 5), (3, 5)))
    q = xp.reshape(N, 4, 58, 4, 58, 4).transpose(0, 2, 4, 3, 5, 1)
    q = q.reshape(N, 58, 58, 64)
    b4 = jnp.concatenate(_stem_phase_weights(b), axis=1)
    return pl.pallas_call(
        functools.partial(_stem_pool_kernel, Ho=Ho, Wo=Wo),
        out_shape=jax.ShapeDtypeStruct((N, Ho + 2, Wo + 2, 64), jnp.bfloat16),
        grid=(N // nb,),
        in_specs=[
            pl.BlockSpec((nb, 58, 58, 64), lambda i: (i, 0, 0, 0)),
            pl.BlockSpec((576, 256), lambda i: (0, 0)),
            pl.BlockSpec((1, 64), lambda i: (0, 0)),
        ],
        out_specs=pl.BlockSpec((nb, Ho + 2, Wo + 2, 64),
                               lambda i: (i, 0, 0, 0)),
        compiler_params=_compiler_params(1),
    )(q, b4, shift)


def avgpool_head(xp, b, shift):
    N, Hp, Wp, C = xp.shape
    out_n = b.shape[1]
    np_ = _cdiv(out_n, 512) * 512
    if np_ != out_n:
        b = jnp.pad(b, ((0, 0), (0, np_ - out_n)))
        shift = jnp.pad(shift, ((0, 0), (0, np_ - out_n)))
    out = pl.pallas_call(
        _head_kernel,
        out_shape=jax.ShapeDtypeStruct((N, np_), jnp.float32),
        grid=(np_ // 512,),
        in_specs=[
            pl.BlockSpec((N, Hp, Wp, C), lambda j: (0, 0, 0, 0)),
            pl.BlockSpec((C, 512), lambda j: (0, j)),
            pl.BlockSpec((1, 512), lambda j: (0, j)),
        ],
        out_specs=pl.BlockSpec((N, 512), lambda j: (0, j)),
        compiler_params=_compiler_params(1),
    )(xp, b, shift)
    return out[:, :out_n] if np_ != out_n else out


def kernel(image, conv1_b, conv1_shift, s0b0_conv1_b, s0b0_conv1_shift, s0b0_conv2_b, s0b0_conv2_shift, s0b1_conv1_b, s0b1_conv1_shift, s0b1_conv2_b, s0b1_conv2_shift, s1b0_conv1_b, s1b0_conv1_shift, s1b0_conv2_b, s1b0_conv2_shift, s1b0_down_b, s1b0_down_shift, s1b1_conv1_b, s1b1_conv1_shift, s1b1_conv2_b, s1b1_conv2_shift, s2b0_conv1_b, s2b0_conv1_shift, s2b0_conv2_b, s2b0_conv2_shift, s2b0_down_b, s2b0_down_shift, s2b1_conv1_b, s2b1_conv1_shift, s2b1_conv2_b, s2b1_conv2_shift, s3b0_conv1_b, s3b0_conv1_shift, s3b0_conv2_b, s3b0_conv2_shift, s3b0_down_b, s3b0_down_shift, s3b1_conv1_b, s3b1_conv1_shift, s3b1_conv2_b, s3b1_conv2_shift, head_b, head_shift):
    x = stem_conv_pool(image, conv1_b, conv1_shift, nb=2)

    x = plain_block(x, s0b0_conv1_b, s0b0_conv1_shift,
                    s0b0_conv2_b, s0b0_conv2_shift, nb=2)
    x = plain_block(x, s0b1_conv1_b, s0b1_conv1_shift,
                    s0b1_conv2_b, s0b1_conv2_shift, nb=2)

    x = down_block(x, s1b0_conv1_b, s1b0_conv1_shift,
                   s1b0_down_b, s1b0_down_shift,
                   s1b0_conv2_b, s1b0_conv2_shift, nb=4)
    x = plain_block(x, s1b1_conv1_b, s1b1_conv1_shift,
                    s1b1_conv2_b, s1b1_conv2_shift, nb=4)

    x = down_block(x, s2b0_conv1_b, s2b0_conv1_shift,
                   s2b0_down_b, s2b0_down_shift,
                   s2b0_conv2_b, s2b0_conv2_shift, nb=8)
    x = plain_block(x, s2b1_conv1_b, s2b1_conv1_shift,
                    s2b1_conv2_b, s2b1_conv2_shift, nb=8)

    x = down_block(x, s3b0_conv1_b, s3b0_conv1_shift,
                   s3b0_down_b, s3b0_down_shift,
                   s3b0_conv2_b, s3b0_conv2_shift, nb=16)
    x = plain_block(x, s3b1_conv1_b, s3b1_conv1_shift,
                    s3b1_conv2_b, s3b1_conv2_shift, nb=16)

    return avgpool_head(x, head_b, head_shift)


# speedup vs baseline: 1.0076x; 1.0076x over previous
import functools

import jax
import jax.numpy as jnp
from jax.experimental import pallas as pl
from jax.experimental.pallas import tpu as pltpu

_VMEM_LIMIT = 56 * 1024 * 1024


def _cdiv(a, b):
    return (a + b - 1) // b


def _pad_ring(y):
    return jnp.pad(y, ((0, 0), (1, 1), (1, 1), (0, 0)))


def _conv3x3(x, b_ref, s_ref, Ho, Wo):
    nb, _, _, C = x.shape
    taps = [x[:, di:di + Ho, dj:dj + Wo, :]
            for di in range(3) for dj in range(3)]
    a = jnp.concatenate(taps, axis=3).reshape(nb * Ho * Wo, 9 * C)
    y = jnp.dot(a, b_ref[...], preferred_element_type=jnp.float32)
    return (y + s_ref[...]).reshape(nb, Ho, Wo, y.shape[-1])


def _plain_block_kernel(x_ref, b1_ref, s1_ref, b2_ref, s2_ref, o_ref,
                        *, Ho, Wo):
    x = x_ref[...]
    y1 = jnp.maximum(_conv3x3(x, b1_ref, s1_ref, Ho, Wo), 0.0)
    mid = _pad_ring(y1.astype(jnp.bfloat16))
    y2 = _conv3x3(mid, b2_ref, s2_ref, Ho, Wo)
    y2 = y2 + x[:, 1:-1, 1:-1, :].astype(jnp.float32)
    y2 = jnp.maximum(y2, 0.0)
    o_ref[...] = _pad_ring(y2.astype(o_ref.dtype))


def _down_block_kernel(x_ref, b1_ref, s1_ref, bd_ref, sd_ref, b2_ref,
                       s2_ref, o_ref, *, C, Ho, Wo):
    x = x_ref[...]
    nb = x.shape[0]
    taps = []
    for di in range(3):
        p, a = di % 2, di // 2
        for dj in range(3):
            q, bb = dj % 2, dj // 2
            taps.append(x[:, a:a + Ho, p, bb:bb + Wo, q * C:(q + 1) * C])
    a_mat = jnp.concatenate(taps, axis=3).reshape(nb * Ho * Wo, 9 * C)
    y1 = jnp.dot(a_mat, b1_ref[...], preferred_element_type=jnp.float32)
    y1 = jnp.maximum(y1 + s1_ref[...], 0.0)
    mid = _pad_ring(y1.reshape(nb, Ho, Wo, y1.shape[-1])
                    .astype(jnp.bfloat16))

    xph = x[:, :Ho, 1, :Wo, C:2 * C]
    yd = jnp.dot(xph.reshape(nb * Ho * Wo, C), bd_ref[...],
                 preferred_element_type=jnp.float32)
    yd = (yd + sd_ref[...]).astype(jnp.bfloat16)

    y2 = _conv3x3(mid, b2_ref, s2_ref, Ho, Wo)
    y2 = y2 + yd.reshape(y2.shape).astype(jnp.float32)
    y2 = jnp.maximum(y2, 0.0)
    o_ref[...] = _pad_ring(y2.astype(o_ref.dtype))


def _stem_pool_kernel(q_ref, b_ref, s_ref, o_ref, *, Ho, Wo):
    q = q_ref[...]
    nb = q.shape[0]
    shift = s_ref[...]
    taps = [q[:, a:a + Ho, b:b + Wo, :]
            for a in range(3) for b in range(3)]
    a_mat = jnp.concatenate(taps, axis=3).reshape(nb * Ho * Wo, 9 * 64)
    y4 = jnp.dot(a_mat, b_ref[...], preferred_element_type=jnp.float32)
    ys = [y4[:, k * 64:(k + 1) * 64].reshape(nb, Ho, Wo, 64)
          for k in range(4)]
    y00, y01, y10, y11 = ys

    def sh_i(y):
        z = jnp.full_like(y[:, :1], -jnp.inf)
        return jnp.concatenate([z, y[:, :-1]], axis=1)

    def sh_j(y):
        z = jnp.full_like(y[:, :, :1], -jnp.inf)
        return jnp.concatenate([z, y[:, :, :-1]], axis=2)

    c0 = jnp.maximum(jnp.maximum(y00, y01), sh_j(y01))
    c1 = jnp.maximum(jnp.maximum(y10, y11), sh_j(y11))
    m = jnp.maximum(jnp.maximum(c0, c1), sh_i(c1))
    m = jnp.maximum(m + shift, 0.0)
    o_ref[...] = _pad_ring(m.astype(o_ref.dtype))


def _head_kernel(x_ref, b_ref, s_ref, o_ref):
    x = x_ref[...][:, 1:-1, 1:-1, :].astype(jnp.float32)
    xm = jnp.mean(x, axis=(1, 2))
    y = jnp.dot(xm.astype(jnp.bfloat16), b_ref[...],
                preferred_element_type=jnp.float32)
    o_ref[...] = y + s_ref[...]


def _compiler_params(n_par):
    return pltpu.CompilerParams(
        dimension_semantics=("parallel",) * n_par,
        vmem_limit_bytes=_VMEM_LIMIT)


def plain_block(xp, b1, s1, b2, s2, *, nb):
    N, Hp, Wp, C = xp.shape
    Ho, Wo = Hp - 2, Wp - 2
    Cout = b2.shape[1]
    return pl.pallas_call(
        functools.partial(_plain_block_kernel, Ho=Ho, Wo=Wo),
        out_shape=jax.ShapeDtypeStruct((N, Ho + 2, Wo + 2, Cout),
                                       jnp.bfloat16),
        grid=(N // nb,),
        in_specs=[
            pl.BlockSpec((nb, Hp, Wp, C), lambda i: (i, 0, 0, 0)),
            pl.BlockSpec((9 * C, Cout), lambda i: (0, 0)),
            pl.BlockSpec((1, Cout), lambda i: (0, 0)),
            pl.BlockSpec((9 * Cout, Cout), lambda i: (0, 0)),
            pl.BlockSpec((1, Cout), lambda i: (0, 0)),
        ],
        out_specs=pl.BlockSpec((nb, Ho + 2, Wo + 2, Cout),
                               lambda i: (i, 0, 0, 0)),
        compiler_params=_compiler_params(1),
    )(xp, b1, s1, b2, s2)


def down_block(xp, b1, s1, bd, sd, b2, s2, *, nb):
    N, Hp, Wp, C = xp.shape
    Ho, Wo = (Hp - 2) // 2, (Wp - 2) // 2
    Cout = b2.shape[1]
    xf = xp.reshape(N, Hp // 2, 2, Wp // 2, 2 * C)
    return pl.pallas_call(
        functools.partial(_down_block_kernel, C=C, Ho=Ho, Wo=Wo),
        out_shape=jax.ShapeDtypeStruct((N, Ho + 2, Wo + 2, Cout),
                                       jnp.bfloat16),
        grid=(N // nb,),
        in_specs=[
            pl.BlockSpec((nb, Hp // 2, 2, Wp // 2, 2 * C),
                         lambda i: (i, 0, 0, 0, 0)),
            pl.BlockSpec((9 * C, Cout), lambda i: (0, 0)),
            pl.BlockSpec((1, Cout), lambda i: (0, 0)),
            pl.BlockSpec((C, Cout), lambda i: (0, 0)),
            pl.BlockSpec((1, Cout), lambda i: (0, 0)),
            pl.BlockSpec((9 * Cout, Cout), lambda i: (0, 0)),
            pl.BlockSpec((1, Cout), lambda i: (0, 0)),
        ],
        out_specs=pl.BlockSpec((nb, Ho + 2, Wo + 2, Cout),
                               lambda i: (i, 0, 0, 0)),
        compiler_params=_compiler_params(1),
    )(xf, b1, s1, bd, sd, b2, s2)


def _stem_phase_weights(b):
    bp = jnp.concatenate([b, jnp.zeros((1, b.shape[1]), b.dtype)], axis=0)
    mats = []
    for r in (0, 1):
        for s in (0, 1):
            rows = []
            for a in range(3):
                for bb in range(3):
                    for gh in range(4):
                        for gw in range(4):
                            for c in range(4):
                                di = 4 * a + gh - 2 * r
                                dj = 4 * bb + gw - 2 * s
                                if 0 <= di < 7 and 0 <= dj < 7 and c < 3:
                                    rows.append((di * 7 + dj) * 3 + c)
                                else:
                                    rows.append(147)
            mats.append(bp[jnp.array(rows)])
    return mats


def stem_conv_pool(image, b, shift, *, nb):
    N = image.shape[0]
    Ho, Wo = 56, 56
    xp = jnp.pad(image.astype(jnp.bfloat16), ((0, 0), (0, 1), (3, 5), (3, 5)))
    q = xp.reshape(N, 4, 58, 4, 58, 4).transpose(0, 2, 4, 3, 5, 1)
    q = q.reshape(N, 58, 58, 64)
    b4 = jnp.concatenate(_stem_phase_weights(b), axis=1)
    return pl.pallas_call(
        functools.partial(_stem_pool_kernel, Ho=Ho, Wo=Wo),
        out_shape=jax.ShapeDtypeStruct((N, Ho + 2, Wo + 2, 64), jnp.bfloat16),
        grid=(N // nb,),
        in_specs=[
            pl.BlockSpec((nb, 58, 58, 64), lambda i: (i, 0, 0, 0)),
            pl.BlockSpec((576, 256), lambda i: (0, 0)),
            pl.BlockSpec((1, 64), lambda i: (0, 0)),
        ],
        out_specs=pl.BlockSpec((nb, Ho + 2, Wo + 2, 64),
                               lambda i: (i, 0, 0, 0)),
        compiler_params=_compiler_params(1),
    )(q, b4, shift)


def avgpool_head(xp, b, shift):
    N, Hp, Wp, C = xp.shape
    out_n = b.shape[1]
    np_ = _cdiv(out_n, 512) * 512
    if np_ != out_n:
        b = jnp.pad(b, ((0, 0), (0, np_ - out_n)))
        shift = jnp.pad(shift, ((0, 0), (0, np_ - out_n)))
    out = pl.pallas_call(
        _head_kernel,
        out_shape=jax.ShapeDtypeStruct((N, np_), jnp.float32),
        grid=(np_ // 512,),
        in_specs=[
            pl.BlockSpec((N, Hp, Wp, C), lambda j: (0, 0, 0, 0)),
            pl.BlockSpec((C, 512), lambda j: (0, j)),
            pl.BlockSpec((1, 512), lambda j: (0, j)),
        ],
        out_specs=pl.BlockSpec((N, 512), lambda j: (0, j)),
        compiler_params=_compiler_params(1),
    )(xp, b, shift)
    return out[:, :out_n] if np_ != out_n else out


def kernel(image, conv1_b, conv1_shift, s0b0_conv1_b, s0b0_conv1_shift, s0b0_conv2_b, s0b0_conv2_shift, s0b1_conv1_b, s0b1_conv1_shift, s0b1_conv2_b, s0b1_conv2_shift, s1b0_conv1_b, s1b0_conv1_shift, s1b0_conv2_b, s1b0_conv2_shift, s1b0_down_b, s1b0_down_shift, s1b1_conv1_b, s1b1_conv1_shift, s1b1_conv2_b, s1b1_conv2_shift, s2b0_conv1_b, s2b0_conv1_shift, s2b0_conv2_b, s2b0_conv2_shift, s2b0_down_b, s2b0_down_shift, s2b1_conv1_b, s2b1_conv1_shift, s2b1_conv2_b, s2b1_conv2_shift, s3b0_conv1_b, s3b0_conv1_shift, s3b0_conv2_b, s3b0_conv2_shift, s3b0_down_b, s3b0_down_shift, s3b1_conv1_b, s3b1_conv1_shift, s3b1_conv2_b, s3b1_conv2_shift, head_b, head_shift):
    x = stem_conv_pool(image, conv1_b, conv1_shift, nb=2)

    x = plain_block(x, s0b0_conv1_b, s0b0_conv1_shift,
                    s0b0_conv2_b, s0b0_conv2_shift, nb=4)
    x = plain_block(x, s0b1_conv1_b, s0b1_conv1_shift,
                    s0b1_conv2_b, s0b1_conv2_shift, nb=4)

    x = down_block(x, s1b0_conv1_b, s1b0_conv1_shift,
                   s1b0_down_b, s1b0_down_shift,
                   s1b0_conv2_b, s1b0_conv2_shift, nb=4)
    x = plain_block(x, s1b1_conv1_b, s1b1_conv1_shift,
                    s1b1_conv2_b, s1b1_conv2_shift, nb=4)

    x = down_block(x, s2b0_conv1_b, s2b0_conv1_shift,
                   s2b0_down_b, s2b0_down_shift,
                   s2b0_conv2_b, s2b0_conv2_shift, nb=8)
    x = plain_block(x, s2b1_conv1_b, s2b1_conv1_shift,
                    s2b1_conv2_b, s2b1_conv2_shift, nb=8)

    x = down_block(x, s3b0_conv1_b, s3b0_conv1_shift,
                   s3b0_down_b, s3b0_down_shift,
                   s3b0_conv2_b, s3b0_conv2_shift, nb=16)
    x = plain_block(x, s3b1_conv1_b, s3b1_conv1_shift,
                    s3b1_conv2_b, s3b1_conv2_shift, nb=16)

    return avgpool_head(x, head_b, head_shift)
